# Initial kernel scaffold; baseline (speedup 1.0000x reference)
import jax
import jax.numpy as jnp
from jax.experimental import pallas as pl


def kernel(x, edge_index, W1l, b1l, W1r, g1, be1, W2l, b2l, W2r, g2, be2, Wp, bp):
    raise NotImplementedError("write your pallas kernel here")


# speedup vs baseline: 17.3604x; 17.3604x over previous
import functools

import jax
import jax.numpy as jnp
from jax import lax
from jax.experimental import pallas as pl
from jax.experimental.pallas import tpu as pltpu
from jax.experimental.pallas import tpu_sc as plsc

N = 100000
E = 6400000
EPS = 1e-5

NSC = 2
NTILES = 16
LANES = 16

ROWS_PER_TILE = 6272
NPAD = NTILES * ROWS_PER_TILE
ZROWS = 128

CH = 1024
CHUNKS = 196
EDGES_PER_TILE = CH * CHUNKS
EDGES_PER_SC = EDGES_PER_TILE * NTILES
E_PAD = EDGES_PER_SC * NSC

_mesh = plsc.VectorSubcoreMesh(core_axis_name="c", subcore_axis_name="s")


def _sc_aggregate(table, srcf, dstf, npasses, scale_idx):
    @functools.partial(
        pl.kernel,
        out_type=jax.ShapeDtypeStruct((NSC, npasses, NPAD, LANES), jnp.float32),
        mesh=_mesh,
        scratch_types=[
            pltpu.VMEM((CH,), jnp.int32),
            pltpu.VMEM((CH,), jnp.int32),
            pltpu.VMEM((CH,), jnp.int32),
            pltpu.VMEM((CH, LANES), jnp.float32),
            pltpu.VMEM((ZROWS, LANES), jnp.float32),
            pltpu.VMEM_SHARED((NPAD, LANES), jnp.float32),
            pltpu.SemaphoreType.DMA,
            pltpu.SemaphoreType.DMA,
        ],
        compiler_params=pltpu.CompilerParams(use_tc_tiling_on_sc=False),
    )
    def agg(table_hbm, src_hbm, dst_hbm, out_hbm,
            src_v, dst_v, gidx_v, rows_v, zero_v, acc_sh, sem_g, sem_s):
        c = lax.axis_index("c")
        s = lax.axis_index("s")

        @pl.loop(0, ZROWS)
        def _(i):
            zero_v[i, :] = jnp.zeros((LANES,), jnp.float32)

        row_base = s * ROWS_PER_TILE
        edge_base = c * EDGES_PER_SC + s * EDGES_PER_TILE

        @pl.loop(0, npasses)
        def _(p):
            @pl.loop(0, ROWS_PER_TILE // ZROWS)
            def _(j):
                pltpu.sync_copy(zero_v,
                                acc_sh.at[pl.ds(row_base + j * ZROWS, ZROWS)])
            plsc.subcore_barrier()

            @pl.loop(0, CHUNKS)
            def _(k):
                eo = edge_base + k * CH
                pltpu.sync_copy(src_hbm.at[pl.ds(eo, CH)], src_v)
                pltpu.sync_copy(dst_hbm.at[pl.ds(eo, CH)], dst_v)
                if scale_idx == 1:
                    gsrc = src_v
                else:
                    @pl.loop(0, CH // LANES)
                    def _(v):
                        sl = pl.ds(v * LANES, LANES)
                        gidx_v[sl] = src_v[sl] * scale_idx + p
                    gsrc = gidx_v
                pltpu.async_copy(table_hbm.at[gsrc], rows_v, sem_g).wait()
                pltpu.async_copy(rows_v, acc_sh.at[dst_v], sem_s,
                                 add=True).wait()

            plsc.subcore_barrier()
            @pl.loop(0, ROWS_PER_TILE // ZROWS)
            def _(j):
                r0 = row_base + j * ZROWS
                pltpu.sync_copy(acc_sh.at[pl.ds(r0, ZROWS)],
                                out_hbm.at[c, p, pl.ds(r0, ZROWS)])
            plsc.subcore_barrier()

    return agg(table, srcf, dstf)


BLK = 512
GRID = NPAD // BLK


def _tc_layer1_body(P_ref, xp_ref, w1l_ref, b1l_ref, w1r_ref, sc1_ref, be1_ref,
                    h1_ref, cnt_ref):
    ssum = P_ref[0] + P_ref[1]
    cnt = jnp.maximum(ssum[:, 4:5], 1.0)
    agg = ssum[:, 0:4] / cnt
    xb = xp_ref[:, 0:4]
    h = (jnp.dot(agg, w1l_ref[...], preferred_element_type=jnp.float32)
         + jnp.dot(xb, w1r_ref[...], preferred_element_type=jnp.float32)
         + b1l_ref[...])
    h = h * sc1_ref[...] + be1_ref[...]
    h1_ref[...] = jnp.maximum(h, 0.0)
    cnt_ref[...] = cnt


def _tc_layer1(P, xp, w1lT, b1l, w1rT, sc1, be1):
    wspec = pl.BlockSpec((4, 64), lambda i: (0, 0))
    vspec = pl.BlockSpec((1, 64), lambda i: (0, 0))
    return pl.pallas_call(
        _tc_layer1_body,
        grid=(GRID,),
        in_specs=[
            pl.BlockSpec((NSC, BLK, LANES), lambda i: (0, i, 0)),
            pl.BlockSpec((BLK, LANES), lambda i: (i, 0)),
            wspec, vspec, wspec, vspec, vspec,
        ],
        out_specs=[
            pl.BlockSpec((BLK, 64), lambda i: (i, 0)),
            pl.BlockSpec((BLK, 1), lambda i: (i, 0)),
        ],
        out_shape=[
            jax.ShapeDtypeStruct((NPAD, 64), jnp.float32),
            jax.ShapeDtypeStruct((NPAD, 1), jnp.float32),
        ],
    )(P[:, 0], xp, w1lT, b1l, w1rT, sc1, be1)


def _tc_layer2_body(O_ref, h1_ref, cnt_ref, w2l_ref, b2l_ref, w2r_ref,
                    sc2_ref, be2_ref, wp_ref, bp_ref, out_ref):
    psum = O_ref[0] + O_ref[1]
    agg = jnp.concatenate([psum[0], psum[1], psum[2], psum[3]], axis=1)
    agg = agg / jnp.maximum(cnt_ref[...], 1.0)
    h1 = h1_ref[...]
    h = (jnp.dot(agg, w2l_ref[...], preferred_element_type=jnp.float32)
         + jnp.dot(h1, w2r_ref[...], preferred_element_type=jnp.float32)
         + b2l_ref[...])
    h = h * sc2_ref[...] + be2_ref[...]
    h = jnp.maximum(h, 0.0)
    o = jnp.dot(h, wp_ref[...], preferred_element_type=jnp.float32) + bp_ref[...]
    out_ref[...] = jax.nn.sigmoid(o)


def _tc_layer2(O, h1, cnt, w2lT, b2l, w2rT, sc2, be2, wpT, bp):
    wspec = pl.BlockSpec((64, 64), lambda i: (0, 0))
    vspec = pl.BlockSpec((1, 64), lambda i: (0, 0))
    return pl.pallas_call(
        _tc_layer2_body,
        grid=(GRID,),
        in_specs=[
            pl.BlockSpec((NSC, 4, BLK, LANES), lambda i: (0, 0, i, 0)),
            pl.BlockSpec((BLK, 64), lambda i: (i, 0)),
            pl.BlockSpec((BLK, 1), lambda i: (i, 0)),
            wspec, vspec, wspec, vspec, vspec,
            pl.BlockSpec((64, 1), lambda i: (0, 0)),
            pl.BlockSpec((1, 1), lambda i: (0, 0)),
        ],
        out_specs=pl.BlockSpec((BLK, 1), lambda i: (i, 0)),
        out_shape=jax.ShapeDtypeStruct((NPAD, 1), jnp.float32),
    )(O, h1, cnt, w2lT, b2l, w2rT, sc2, be2, wpT, bp)


def kernel(x, edge_index, W1l, b1l, W1r, g1, be1, W2l, b2l, W2r, g2, be2, Wp, bp):
    src = edge_index[0]
    dst = edge_index[1]

    pad = E_PAD - E
    padidx = (jnp.arange(pad, dtype=jnp.int32) % 128) + N
    srcf = jnp.concatenate([src, padidx])
    dstf = jnp.concatenate([dst, padidx])

    xp = jnp.concatenate(
        [x, jnp.ones((N, 1), jnp.float32), jnp.zeros((N, LANES - 5), jnp.float32)],
        axis=1)
    xp = jnp.pad(xp, ((0, NPAD - N), (0, 0)))

    sc1 = (g1 / jnp.sqrt(1.0 + EPS)).reshape(1, 64)
    sc2 = (g2 / jnp.sqrt(1.0 + EPS)).reshape(1, 64)

    P = _sc_aggregate(xp, srcf, dstf, npasses=1, scale_idx=1)
    h1, cnt = _tc_layer1(P, xp, W1l.T, b1l.reshape(1, 64), W1r.T,
                         sc1, be1.reshape(1, 64))

    h4 = h1.reshape(4 * NPAD, LANES)
    O = _sc_aggregate(h4, srcf, dstf, npasses=4, scale_idx=4)
    out = _tc_layer2(O, h1, cnt, W2l.T, b2l.reshape(1, 64), W2r.T,
                     sc2, be2.reshape(1, 64), Wp.T, bp.reshape(1, 1))
    return out[:N]


# speedup vs baseline: 21.8237x; 1.2571x over previous
import functools

import jax
import jax.numpy as jnp
from jax import lax
from jax.experimental import pallas as pl
from jax.experimental.pallas import tpu as pltpu
from jax.experimental.pallas import tpu_sc as plsc

N = 100000
E = 6400000
EPS = 1e-5

NSC = 2
NTILES = 16
LANES = 16

ROWS_PER_TILE = 6272
NPAD = NTILES * ROWS_PER_TILE
ZROWS = 128

CH = 512
CHUNKS = 392
EDGES_PER_TILE = CH * CHUNKS
EDGES_PER_SC = EDGES_PER_TILE * NTILES
E_PAD = EDGES_PER_SC * NSC

_mesh = plsc.VectorSubcoreMesh(core_axis_name="c", subcore_axis_name="s")


def _sc_aggregate(table, srcf, dstf, npasses, scale_idx):
    @functools.partial(
        pl.kernel,
        out_type=jax.ShapeDtypeStruct((NSC, npasses, NPAD, LANES), jnp.float32),
        mesh=_mesh,
        scratch_types=[
            pltpu.VMEM((4, CH), jnp.int32),
            pltpu.VMEM((4, CH), jnp.int32),
            pltpu.VMEM((2, CH, LANES), jnp.float32),
            pltpu.VMEM((ZROWS, LANES), jnp.float32),
            pltpu.VMEM_SHARED((NPAD, LANES), jnp.float32),
            pltpu.SemaphoreType.DMA,
            pltpu.SemaphoreType.DMA,
            pltpu.SemaphoreType.DMA,
        ],
        compiler_params=pltpu.CompilerParams(use_tc_tiling_on_sc=False),
    )
    def agg(table_hbm, src_hbm, dst_hbm, out_hbm,
            src_v, dst_v, rows_v, zero_v, acc_sh, sem_i, sem_g, sem_s):
        c = lax.axis_index("c")
        s = lax.axis_index("s")

        @pl.loop(0, ZROWS)
        def _(i):
            zero_v[i, :] = jnp.zeros((LANES,), jnp.float32)

        row_base = s * ROWS_PER_TILE
        edge_base = c * EDGES_PER_SC + s * EDGES_PER_TILE

        def issue_idx(j, u):
            eo = edge_base + j * CH
            pltpu.async_copy(src_hbm.at[pl.ds(eo, CH)], src_v.at[u], sem_i)
            pltpu.async_copy(dst_hbm.at[pl.ds(eo, CH)], dst_v.at[u], sem_i)

        def wait_idx(j, u):
            eo = edge_base + j * CH
            pltpu.make_async_copy(src_hbm.at[pl.ds(eo, CH)], src_v.at[u],
                                  sem_i).wait()
            pltpu.make_async_copy(dst_hbm.at[pl.ds(eo, CH)], dst_v.at[u],
                                  sem_i).wait()

        def wait_scatter(b, u):
            pltpu.make_async_copy(rows_v.at[b], acc_sh.at[dst_v.at[u]],
                                  sem_s).wait()

        @pl.loop(0, npasses)
        def _(p):
            @pl.loop(0, ROWS_PER_TILE // ZROWS)
            def _(j):
                pltpu.sync_copy(zero_v,
                                acc_sh.at[pl.ds(row_base + j * ZROWS, ZROWS)])
            plsc.subcore_barrier()

            issue_idx(0, 0)

            @pl.loop(0, CHUNKS // 4)
            def _(t):
                for u in range(4):
                    j = t * 4 + u
                    b = u % 2
                    wait_idx(j, u)
                    if scale_idx != 1:
                        @pl.loop(0, CH // LANES)
                        def _(v):
                            sl = pl.ds(v * LANES, LANES)
                            src_v[u, sl] = src_v[u, sl] * scale_idx + p
                    if u >= 2:
                        wait_scatter(b, u - 2)
                    else:
                        @pl.when(t > 0)
                        def _():
                            wait_scatter(b, u + 2)
                    g = pltpu.async_copy(table_hbm.at[src_v.at[u]],
                                         rows_v.at[b], sem_g)
                    if u < 3:
                        issue_idx(j + 1, u + 1)
                    else:
                        @pl.when(t < CHUNKS // 4 - 1)
                        def _():
                            issue_idx(j + 1, 0)
                    g.wait()
                    pltpu.async_copy(rows_v.at[b], acc_sh.at[dst_v.at[u]],
                                     sem_s, add=True)

            wait_scatter(0, 2)
            wait_scatter(1, 3)

            plsc.subcore_barrier()
            @pl.loop(0, ROWS_PER_TILE // ZROWS)
            def _(j):
                r0 = row_base + j * ZROWS
                pltpu.sync_copy(acc_sh.at[pl.ds(r0, ZROWS)],
                                out_hbm.at[c, p, pl.ds(r0, ZROWS)])
            plsc.subcore_barrier()

    return agg(table, srcf, dstf)


BLK = 512
GRID = NPAD // BLK


def _tc_layer1_body(P_ref, xp_ref, w1l_ref, b1l_ref, w1r_ref, sc1_ref, be1_ref,
                    h1_ref, cnt_ref):
    ssum = P_ref[0] + P_ref[1]
    cnt = jnp.maximum(ssum[:, 4:5], 1.0)
    agg = ssum[:, 0:4] / cnt
    xb = xp_ref[:, 0:4]
    h = (jnp.dot(agg, w1l_ref[...], preferred_element_type=jnp.float32)
         + jnp.dot(xb, w1r_ref[...], preferred_element_type=jnp.float32)
         + b1l_ref[...])
    h = h * sc1_ref[...] + be1_ref[...]
    h1_ref[...] = jnp.maximum(h, 0.0)
    cnt_ref[...] = cnt


def _tc_layer1(P, xp, w1lT, b1l, w1rT, sc1, be1):
    wspec = pl.BlockSpec((4, 64), lambda i: (0, 0))
    vspec = pl.BlockSpec((1, 64), lambda i: (0, 0))
    return pl.pallas_call(
        _tc_layer1_body,
        grid=(GRID,),
        in_specs=[
            pl.BlockSpec((NSC, BLK, LANES), lambda i: (0, i, 0)),
            pl.BlockSpec((BLK, LANES), lambda i: (i, 0)),
            wspec, vspec, wspec, vspec, vspec,
        ],
        out_specs=[
            pl.BlockSpec((BLK, 64), lambda i: (i, 0)),
            pl.BlockSpec((BLK, 1), lambda i: (i, 0)),
        ],
        out_shape=[
            jax.ShapeDtypeStruct((NPAD, 64), jnp.float32),
            jax.ShapeDtypeStruct((NPAD, 1), jnp.float32),
        ],
    )(P[:, 0], xp, w1lT, b1l, w1rT, sc1, be1)


def _tc_layer2_body(O_ref, h1_ref, cnt_ref, w2l_ref, b2l_ref, w2r_ref,
                    sc2_ref, be2_ref, wp_ref, bp_ref, out_ref):
    psum = O_ref[0] + O_ref[1]
    agg = jnp.concatenate([psum[0], psum[1], psum[2], psum[3]], axis=1)
    agg = agg / jnp.maximum(cnt_ref[...], 1.0)
    h1 = h1_ref[...]
    h = (jnp.dot(agg, w2l_ref[...], preferred_element_type=jnp.float32)
         + jnp.dot(h1, w2r_ref[...], preferred_element_type=jnp.float32)
         + b2l_ref[...])
    h = h * sc2_ref[...] + be2_ref[...]
    h = jnp.maximum(h, 0.0)
    o = jnp.dot(h, wp_ref[...], preferred_element_type=jnp.float32) + bp_ref[...]
    out_ref[...] = jax.nn.sigmoid(o)


def _tc_layer2(O, h1, cnt, w2lT, b2l, w2rT, sc2, be2, wpT, bp):
    wspec = pl.BlockSpec((64, 64), lambda i: (0, 0))
    vspec = pl.BlockSpec((1, 64), lambda i: (0, 0))
    return pl.pallas_call(
        _tc_layer2_body,
        grid=(GRID,),
        in_specs=[
            pl.BlockSpec((NSC, 4, BLK, LANES), lambda i: (0, 0, i, 0)),
            pl.BlockSpec((BLK, 64), lambda i: (i, 0)),
            pl.BlockSpec((BLK, 1), lambda i: (i, 0)),
            wspec, vspec, wspec, vspec, vspec,
            pl.BlockSpec((64, 1), lambda i: (0, 0)),
            pl.BlockSpec((1, 1), lambda i: (0, 0)),
        ],
        out_specs=pl.BlockSpec((BLK, 1), lambda i: (i, 0)),
        out_shape=jax.ShapeDtypeStruct((NPAD, 1), jnp.float32),
    )(O, h1, cnt, w2lT, b2l, w2rT, sc2, be2, wpT, bp)


def kernel(x, edge_index, W1l, b1l, W1r, g1, be1, W2l, b2l, W2r, g2, be2, Wp, bp):
    src = edge_index[0]
    dst = edge_index[1]

    pad = E_PAD - E
    padidx = (jnp.arange(pad, dtype=jnp.int32) % 128) + N
    srcf = jnp.concatenate([src, padidx])
    dstf = jnp.concatenate([dst, padidx])

    xp = jnp.concatenate(
        [x, jnp.ones((N, 1), jnp.float32), jnp.zeros((N, LANES - 5), jnp.float32)],
        axis=1)
    xp = jnp.pad(xp, ((0, NPAD - N), (0, 0)))

    sc1 = (g1 / jnp.sqrt(1.0 + EPS)).reshape(1, 64)
    sc2 = (g2 / jnp.sqrt(1.0 + EPS)).reshape(1, 64)

    P = _sc_aggregate(xp, srcf, dstf, npasses=1, scale_idx=1)
    h1, cnt = _tc_layer1(P, xp, W1l.T, b1l.reshape(1, 64), W1r.T,
                         sc1, be1.reshape(1, 64))

    h4 = h1.reshape(4 * NPAD, LANES)
    O = _sc_aggregate(h4, srcf, dstf, npasses=4, scale_idx=4)
    out = _tc_layer2(O, h1, cnt, W2l.T, b2l.reshape(1, 64), W2r.T,
                     sc2, be2.reshape(1, 64), Wp.T, bp.reshape(1, 1))
    return out[:N]


# speedup vs baseline: 24.0038x; 1.0999x over previous
import dataclasses
import functools

import jax
import jax.numpy as jnp
from jax import lax
from jax.experimental import pallas as pl
from jax.experimental.pallas import tpu as pltpu
from jax.experimental.pallas import tpu_sc as plsc

N = 100000
E = 6400000
EPS = 1e-5

NSC = 2
NTILES = 16
LANES = 16

ROWS_PER_TILE = 6272
NPAD = NTILES * ROWS_PER_TILE
ZROWS = 128
TROW = 64

CH = 400
CHUNKS = 500
EDGES_PER_TILE = CH * CHUNKS
EDGES_PER_SC = EDGES_PER_TILE * NTILES

_mesh = plsc.VectorSubcoreMesh(core_axis_name="c", subcore_axis_name="s")

_cparams = pltpu.CompilerParams(use_tc_tiling_on_sc=False)


def _chunk_pipeline(src_hbm, dst_hbm, gather_ref, src_v, dst_v, rows_v,
                    acc_sh, sem_i, sem_g, sem_s, edge_base, ch, chunks,
                    idx_mulp=None):
    def issue_idx(j, u):
        eo = edge_base + j * ch
        pltpu.async_copy(src_hbm.at[pl.ds(eo, ch)], src_v.at[u], sem_i)
        pltpu.async_copy(dst_hbm.at[pl.ds(eo, ch)], dst_v.at[u], sem_i)

    def wait_idx(j, u):
        eo = edge_base + j * ch
        pltpu.make_async_copy(src_hbm.at[pl.ds(eo, ch)], src_v.at[u],
                              sem_i).wait()
        pltpu.make_async_copy(dst_hbm.at[pl.ds(eo, ch)], dst_v.at[u],
                              sem_i).wait()

    def wait_scatter(b, u):
        pltpu.make_async_copy(rows_v.at[b], acc_sh.at[dst_v.at[u]],
                              sem_s).wait()

    issue_idx(0, 0)

    @pl.loop(0, chunks // 4)
    def _(t):
        for u in range(4):
            j = t * 4 + u
            b = u % 2
            wait_idx(j, u)
            if idx_mulp is not None:
                @pl.loop(0, ch // LANES)
                def _(v):
                    sl = pl.ds(v * LANES, LANES)
                    src_v[u, sl] = src_v[u, sl] * 4 + idx_mulp
            if u >= 2:
                wait_scatter(b, u - 2)
            else:
                @pl.when(t > 0)
                def _():
                    wait_scatter(b, u + 2)
            g = pltpu.async_copy(gather_ref.at[src_v.at[u]],
                                 rows_v.at[b], sem_g)
            if u < 3:
                issue_idx(j + 1, u + 1)
            else:
                @pl.when(t < chunks // 4 - 1)
                def _():
                    issue_idx(j + 1, 0)
            g.wait()
            pltpu.async_copy(rows_v.at[b], acc_sh.at[dst_v.at[u]],
                             sem_s, add=True)

    wait_scatter(0, 2)
    wait_scatter(1, 3)


W1 = 8
CH1 = 2000
CHUNKS1 = 100


def _sc_layer1(x8, srcf, dstf):
    @functools.partial(
        pl.kernel,
        out_type=jax.ShapeDtypeStruct((NSC, NPAD, W1), jnp.float32),
        mesh=_mesh,
        scratch_types=[
            pltpu.VMEM((4, CH1), jnp.int32),
            pltpu.VMEM((4, CH1), jnp.int32),
            pltpu.VMEM((2, CH1, W1), jnp.float32),
            pltpu.VMEM((ZROWS, W1), jnp.float32),
            pltpu.VMEM_SHARED((NPAD, W1), jnp.float32),
            pltpu.SemaphoreType.DMA,
            pltpu.SemaphoreType.DMA,
            pltpu.SemaphoreType.DMA,
        ],
        compiler_params=_cparams,
    )
    def k(x8_hbm, src_hbm, dst_hbm, out_hbm,
          src_v, dst_v, rows_v, zero_v, acc_sh, sem_i, sem_g, sem_s):
        c = lax.axis_index("c")
        s = lax.axis_index("s")
        row_base = s * ROWS_PER_TILE
        edge_base = c * EDGES_PER_SC + s * EDGES_PER_TILE

        @pl.loop(0, ZROWS * W1 // LANES)
        def _(i):
            zero_v[pl.ds(i * 2, 2), :] = jnp.zeros((2, W1), jnp.float32)

        @pl.loop(0, ROWS_PER_TILE // ZROWS)
        def _(j):
            pltpu.sync_copy(zero_v,
                            acc_sh.at[pl.ds(row_base + j * ZROWS, ZROWS)])
        plsc.subcore_barrier()

        _chunk_pipeline(src_hbm, dst_hbm, x8_hbm, src_v, dst_v,
                        rows_v, acc_sh, sem_i, sem_g, sem_s, edge_base,
                        CH1, CHUNKS1)

        plsc.subcore_barrier()
        @pl.loop(0, ROWS_PER_TILE // ZROWS)
        def _(j):
            r0 = row_base + j * ZROWS
            pltpu.sync_copy(acc_sh.at[pl.ds(r0, ZROWS)],
                            out_hbm.at[c, pl.ds(r0, ZROWS)])

    return k(x8, srcf, dstf)


def _sc_layer2(h4, srcf, dstf):
    @functools.partial(
        pl.kernel,
        out_type=jax.ShapeDtypeStruct((NSC, NPAD, 64), jnp.float32),
        mesh=_mesh,
        scratch_types=[
            pltpu.VMEM((4, CH), jnp.int32),
            pltpu.VMEM((4, CH), jnp.int32),
            pltpu.VMEM((2, CH, LANES), jnp.float32),
            pltpu.VMEM((ZROWS, LANES), jnp.float32),
            pltpu.VMEM_SHARED((NPAD, LANES), jnp.float32),
            pltpu.SemaphoreType.DMA,
            pltpu.SemaphoreType.DMA,
            pltpu.SemaphoreType.DMA,
        ],
        compiler_params=_cparams,
    )
    def k(h4_hbm, src_hbm, dst_hbm, out_hbm,
          src_v, dst_v, rows_v, zero_v, acc_sh, sem_i, sem_g, sem_s):
        c = lax.axis_index("c")
        s = lax.axis_index("s")
        row_base = s * ROWS_PER_TILE
        edge_base = c * EDGES_PER_SC + s * EDGES_PER_TILE

        @pl.loop(0, ZROWS)
        def _(i):
            zero_v[i, :] = jnp.zeros((LANES,), jnp.float32)

        @pl.loop(0, 4)
        def _(p):
            @pl.loop(0, ROWS_PER_TILE // ZROWS)
            def _(j):
                pltpu.sync_copy(zero_v,
                                acc_sh.at[pl.ds(row_base + j * ZROWS, ZROWS)])
            plsc.subcore_barrier()

            _chunk_pipeline(src_hbm, dst_hbm, h4_hbm,
                            src_v, dst_v, rows_v, acc_sh,
                            sem_i, sem_g, sem_s, edge_base, CH, CHUNKS,
                            idx_mulp=p)

            plsc.subcore_barrier()
            @pl.loop(0, ROWS_PER_TILE // ZROWS)
            def _(j):
                r0 = row_base + j * ZROWS
                pltpu.sync_copy(
                    acc_sh.at[pl.ds(r0, ZROWS)],
                    out_hbm.at[c, pl.ds(r0, ZROWS), pl.ds(p * LANES, LANES)])
            plsc.subcore_barrier()

    return k(h4, srcf, dstf)


BLK = 2048
GRID = NPAD // BLK


def _tc_layer1_body(P_ref, x_ref, w1l_ref, b1l_ref, w1r_ref, sc1_ref, be1_ref,
                    h1_ref):
    ssum = P_ref[0] + P_ref[1]
    cnt = jnp.maximum(ssum[:, 4:5], 1.0)
    agg = ssum[:, 0:4] / cnt
    h = (jnp.dot(agg, w1l_ref[...], preferred_element_type=jnp.float32)
         + jnp.dot(x_ref[...], w1r_ref[...], preferred_element_type=jnp.float32)
         + b1l_ref[...])
    h = h * sc1_ref[...] + be1_ref[...]
    h1_ref[...] = jnp.maximum(h, 0.0)


def _tc_layer1(P, x_pad, w1lT, b1l, w1rT, sc1, be1):
    wspec = pl.BlockSpec((4, 64), lambda i: (0, 0))
    vspec = pl.BlockSpec((1, 64), lambda i: (0, 0))
    return pl.pallas_call(
        _tc_layer1_body,
        grid=(GRID,),
        in_specs=[
            pl.BlockSpec((NSC, BLK, 8), lambda i: (0, i, 0)),
            pl.BlockSpec((BLK, 4), lambda i: (i, 0)),
            wspec, vspec, wspec, vspec, vspec,
        ],
        out_specs=pl.BlockSpec((BLK, 64), lambda i: (i, 0)),
        out_shape=jax.ShapeDtypeStruct((NPAD, 64), jnp.float32),
    )(P, x_pad, w1lT, b1l, w1rT, sc1, be1)


def _tc_layer2_body(O_ref, h1_ref, P_ref, w2l_ref, b2l_ref, w2r_ref,
                    sc2_ref, be2_ref, wp_ref, bp_ref, out_ref):
    cnt = jnp.maximum(P_ref[0][:, 4:5] + P_ref[1][:, 4:5], 1.0)
    agg = (O_ref[0] + O_ref[1]) / cnt
    h1 = h1_ref[...]
    h = (jnp.dot(agg, w2l_ref[...], preferred_element_type=jnp.float32)
         + jnp.dot(h1, w2r_ref[...], preferred_element_type=jnp.float32)
         + b2l_ref[...])
    h = h * sc2_ref[...] + be2_ref[...]
    h = jnp.maximum(h, 0.0)
    o = jnp.dot(h, wp_ref[...], preferred_element_type=jnp.float32) + bp_ref[...]
    out_ref[...] = jax.nn.sigmoid(o[:, 0])


def _tc_layer2(O, h1, P, w2lT, b2l, w2rT, sc2, be2, wpT, bp):
    wspec = pl.BlockSpec((64, 64), lambda i: (0, 0))
    vspec = pl.BlockSpec((1, 64), lambda i: (0, 0))
    return pl.pallas_call(
        _tc_layer2_body,
        grid=(GRID,),
        in_specs=[
            pl.BlockSpec((NSC, BLK, 64), lambda i: (0, i, 0)),
            pl.BlockSpec((BLK, 64), lambda i: (i, 0)),
            pl.BlockSpec((NSC, BLK, 8), lambda i: (0, i, 0)),
            wspec, vspec, wspec, vspec, vspec,
            pl.BlockSpec((64, 1), lambda i: (0, 0)),
            pl.BlockSpec((1, 1), lambda i: (0, 0)),
        ],
        out_specs=pl.BlockSpec((BLK,), lambda i: (i,)),
        out_shape=jax.ShapeDtypeStruct((NPAD,), jnp.float32),
    )(O, h1, P, w2lT, b2l, w2rT, sc2, be2, wpT, bp)


def kernel(x, edge_index, W1l, b1l, W1r, g1, be1, W2l, b2l, W2r, g2, be2, Wp, bp):
    src = edge_index[0]
    dst = edge_index[1]

    x8 = jnp.concatenate(
        [x, jnp.ones((N, 1), jnp.float32), jnp.zeros((N, 3), jnp.float32)],
        axis=1)
    x8 = jnp.pad(x8, ((0, NPAD - N), (0, 0)))
    x_pad = jnp.pad(x, ((0, NPAD - N), (0, 0)))

    sc1 = (g1 / jnp.sqrt(1.0 + EPS)).reshape(1, 64)
    sc2 = (g2 / jnp.sqrt(1.0 + EPS)).reshape(1, 64)

    P = _sc_layer1(x8, src, dst)
    h1 = _tc_layer1(P, x_pad, W1l.T, b1l.reshape(1, 64), W1r.T,
                    sc1, be1.reshape(1, 64))
    O = _sc_layer2(h1.reshape(4 * NPAD, LANES), src, dst)
    out = _tc_layer2(O, h1, P, W2l.T, b2l.reshape(1, 64), W2r.T,
                     sc2, be2.reshape(1, 64), Wp.T, bp.reshape(1, 1))
    return out[:N].reshape(N, 1)


# speedup vs baseline: 30.1069x; 1.2543x over previous
import dataclasses
import functools

import jax
import jax.numpy as jnp
from jax import lax
from jax.experimental import pallas as pl
from jax.experimental.pallas import tpu as pltpu
from jax.experimental.pallas import tpu_sc as plsc

N = 100000
E = 6400000
EPS = 1e-5

NSC = 2
NTILES = 16
LANES = 16

ROWS_PER_TILE = 6272
NPAD = NTILES * ROWS_PER_TILE
ZROWS = 128
TROW = 64

CH = 400
CHUNKS = 500
EDGES_PER_TILE = CH * CHUNKS
EDGES_PER_SC = EDGES_PER_TILE * NTILES

_mesh = plsc.VectorSubcoreMesh(core_axis_name="c", subcore_axis_name="s")

_cparams = pltpu.CompilerParams(use_tc_tiling_on_sc=False)


def _chunk_pipeline(ei_hbm, gather_ref, src_v, dst_v, rows_v,
                    acc_sh, sem_i, sem_g, sem_s, edge_base, ch, chunks,
                    idx_mulp=None):
    def issue_idx(j, u):
        eo = edge_base + j * ch
        pltpu.async_copy(ei_hbm.at[0, pl.ds(eo, ch)], src_v.at[u], sem_i)
        pltpu.async_copy(ei_hbm.at[1, pl.ds(eo, ch)], dst_v.at[u], sem_i)

    def wait_idx(j, u):
        eo = edge_base + j * ch
        pltpu.make_async_copy(ei_hbm.at[0, pl.ds(eo, ch)], src_v.at[u],
                              sem_i).wait()
        pltpu.make_async_copy(ei_hbm.at[1, pl.ds(eo, ch)], dst_v.at[u],
                              sem_i).wait()

    def wait_gather(b, u):
        pltpu.make_async_copy(gather_ref.at[src_v.at[u]], rows_v.at[b],
                              sem_g).wait()

    def wait_scatter(b, u):
        pltpu.make_async_copy(rows_v.at[b], acc_sh.at[dst_v.at[u]],
                              sem_s).wait()

    issue_idx(0, 0)

    @pl.loop(0, chunks)
    def _(j):
        u = lax.rem(j, 4)
        b = lax.rem(j, 3)
        up = lax.rem(j + 3, 4)
        bp = lax.rem(j + 2, 3)
        wait_idx(j, u)
        if idx_mulp is not None:
            for v in range(ch // LANES):
                sl = pl.ds(v * LANES, LANES)
                src_v[u, sl] = src_v[u, sl] * 4 + idx_mulp

        @pl.when(j >= 3)
        def _():
            wait_scatter(b, lax.rem(j + 1, 4))
        pltpu.async_copy(gather_ref.at[src_v.at[u]], rows_v.at[b], sem_g)

        @pl.when(j + 1 < chunks)
        def _():
            issue_idx(j + 1, lax.rem(j + 1, 4))

        @pl.when(j >= 1)
        def _():
            wait_gather(bp, up)
            pltpu.async_copy(rows_v.at[bp], acc_sh.at[dst_v.at[up]],
                             sem_s, add=True)

    jl = chunks - 1
    wait_gather(jl % 3, jl % 4)
    pltpu.async_copy(rows_v.at[jl % 3], acc_sh.at[dst_v.at[jl % 4]],
                     sem_s, add=True)
    for jj in (chunks - 3, chunks - 2, chunks - 1):
        wait_scatter(jj % 3, jj % 4)


W1 = 8
CH1 = 2000
CHUNKS1 = 100


def _sc_layer1(x8, ei):
    @functools.partial(
        pl.kernel,
        out_type=jax.ShapeDtypeStruct((NSC, NPAD, W1), jnp.float32),
        mesh=_mesh,
        scratch_types=[
            pltpu.VMEM((4, CH1), jnp.int32),
            pltpu.VMEM((4, CH1), jnp.int32),
            pltpu.VMEM((3, CH1, W1), jnp.float32),
            pltpu.VMEM((ZROWS, W1), jnp.float32),
            pltpu.VMEM_SHARED((NPAD, W1), jnp.float32),
            pltpu.SemaphoreType.DMA,
            pltpu.SemaphoreType.DMA,
            pltpu.SemaphoreType.DMA,
        ],
        compiler_params=_cparams,
    )
    def k(x8_hbm, ei_hbm, out_hbm,
          src_v, dst_v, rows_v, zero_v, acc_sh, sem_i, sem_g, sem_s):
        c = lax.axis_index("c")
        s = lax.axis_index("s")
        row_base = s * ROWS_PER_TILE
        edge_base = c * EDGES_PER_SC + s * EDGES_PER_TILE

        @pl.loop(0, ZROWS * W1 // LANES)
        def _(i):
            zero_v[pl.ds(i * 2, 2), :] = jnp.zeros((2, W1), jnp.float32)

        @pl.loop(0, ROWS_PER_TILE // ZROWS)
        def _(j):
            pltpu.sync_copy(zero_v,
                            acc_sh.at[pl.ds(row_base + j * ZROWS, ZROWS)])
        plsc.subcore_barrier()

        _chunk_pipeline(ei_hbm, x8_hbm, src_v, dst_v,
                        rows_v, acc_sh, sem_i, sem_g, sem_s, edge_base,
                        CH1, CHUNKS1)

        plsc.subcore_barrier()
        @pl.loop(0, ROWS_PER_TILE // ZROWS)
        def _(j):
            r0 = row_base + j * ZROWS
            pltpu.sync_copy(acc_sh.at[pl.ds(r0, ZROWS)],
                            out_hbm.at[c, pl.ds(r0, ZROWS)])

    return k(x8, ei)


def _sc_layer2(h4, ei):
    @functools.partial(
        pl.kernel,
        out_type=jax.ShapeDtypeStruct((NSC, NPAD, 64), jnp.float32),
        mesh=_mesh,
        scratch_types=[
            pltpu.VMEM((4, CH), jnp.int32),
            pltpu.VMEM((4, CH), jnp.int32),
            pltpu.VMEM((3, CH, LANES), jnp.float32),
            pltpu.VMEM((ZROWS, LANES), jnp.float32),
            pltpu.VMEM_SHARED((NPAD, LANES), jnp.float32),
            pltpu.SemaphoreType.DMA,
            pltpu.SemaphoreType.DMA,
            pltpu.SemaphoreType.DMA,
        ],
        compiler_params=_cparams,
    )
    def k(h4_hbm, ei_hbm, out_hbm,
          src_v, dst_v, rows_v, zero_v, acc_sh, sem_i, sem_g, sem_s):
        c = lax.axis_index("c")
        s = lax.axis_index("s")
        row_base = s * ROWS_PER_TILE
        edge_base = c * EDGES_PER_SC + s * EDGES_PER_TILE

        @pl.loop(0, ZROWS)
        def _(i):
            zero_v[i, :] = jnp.zeros((LANES,), jnp.float32)

        @pl.loop(0, 4)
        def _(p):
            @pl.loop(0, ROWS_PER_TILE // ZROWS)
            def _(j):
                pltpu.sync_copy(zero_v,
                                acc_sh.at[pl.ds(row_base + j * ZROWS, ZROWS)])
            plsc.subcore_barrier()

            _chunk_pipeline(ei_hbm, h4_hbm,
                            src_v, dst_v, rows_v, acc_sh,
                            sem_i, sem_g, sem_s, edge_base, CH, CHUNKS,
                            idx_mulp=p)

            plsc.subcore_barrier()
            @pl.loop(0, ROWS_PER_TILE // ZROWS)
            def _(j):
                r0 = row_base + j * ZROWS
                pltpu.sync_copy(
                    acc_sh.at[pl.ds(r0, ZROWS)],
                    out_hbm.at[c, pl.ds(r0, ZROWS), pl.ds(p * LANES, LANES)])
            plsc.subcore_barrier()

    return k(h4, ei)


BLK = 2048
GRID = NPAD // BLK


def _tc_layer1_body(P_ref, x_ref, w1l_ref, b1l_ref, w1r_ref, sc1_ref, be1_ref,
                    h1_ref):
    ssum = P_ref[0] + P_ref[1]
    cnt = jnp.maximum(ssum[:, 4:5], 1.0)
    agg = ssum[:, 0:4] / cnt
    h = (jnp.dot(agg, w1l_ref[...], preferred_element_type=jnp.float32)
         + jnp.dot(x_ref[...], w1r_ref[...], preferred_element_type=jnp.float32)
         + b1l_ref[...])
    h = h * sc1_ref[...] + be1_ref[...]
    h1_ref[...] = jnp.maximum(h, 0.0)


def _tc_layer1(P, x_pad, w1lT, b1l, w1rT, sc1, be1):
    wspec = pl.BlockSpec((4, 64), lambda i: (0, 0))
    vspec = pl.BlockSpec((1, 64), lambda i: (0, 0))
    return pl.pallas_call(
        _tc_layer1_body,
        grid=(GRID,),
        in_specs=[
            pl.BlockSpec((NSC, BLK, 8), lambda i: (0, i, 0)),
            pl.BlockSpec((BLK, 4), lambda i: (i, 0)),
            wspec, vspec, wspec, vspec, vspec,
        ],
        out_specs=pl.BlockSpec((BLK, 64), lambda i: (i, 0)),
        out_shape=jax.ShapeDtypeStruct((NPAD, 64), jnp.float32),
    )(P, x_pad, w1lT, b1l, w1rT, sc1, be1)


def _tc_layer2_body(O_ref, h1_ref, P_ref, w2l_ref, b2l_ref, w2r_ref,
                    sc2_ref, be2_ref, wp_ref, bp_ref, out_ref):
    cnt = jnp.maximum(P_ref[0][:, 4:5] + P_ref[1][:, 4:5], 1.0)
    agg = (O_ref[0] + O_ref[1]) / cnt
    h1 = h1_ref[...]
    h = (jnp.dot(agg, w2l_ref[...], preferred_element_type=jnp.float32)
         + jnp.dot(h1, w2r_ref[...], preferred_element_type=jnp.float32)
         + b2l_ref[...])
    h = h * sc2_ref[...] + be2_ref[...]
    h = jnp.maximum(h, 0.0)
    o = jnp.dot(h, wp_ref[...], preferred_element_type=jnp.float32) + bp_ref[...]
    out_ref[...] = jax.nn.sigmoid(o[:, 0])


def _tc_layer2(O, h1, P, w2lT, b2l, w2rT, sc2, be2, wpT, bp):
    wspec = pl.BlockSpec((64, 64), lambda i: (0, 0))
    vspec = pl.BlockSpec((1, 64), lambda i: (0, 0))
    return pl.pallas_call(
        _tc_layer2_body,
        grid=(GRID,),
        in_specs=[
            pl.BlockSpec((NSC, BLK, 64), lambda i: (0, i, 0)),
            pl.BlockSpec((BLK, 64), lambda i: (i, 0)),
            pl.BlockSpec((NSC, BLK, 8), lambda i: (0, i, 0)),
            wspec, vspec, wspec, vspec, vspec,
            pl.BlockSpec((64, 1), lambda i: (0, 0)),
            pl.BlockSpec((1, 1), lambda i: (0, 0)),
        ],
        out_specs=pl.BlockSpec((BLK,), lambda i: (i,)),
        out_shape=jax.ShapeDtypeStruct((NPAD,), jnp.float32),
    )(O, h1, P, w2lT, b2l, w2rT, sc2, be2, wpT, bp)


def kernel(x, edge_index, W1l, b1l, W1r, g1, be1, W2l, b2l, W2r, g2, be2, Wp, bp):
    x8 = jnp.concatenate(
        [x, jnp.ones((N, 1), jnp.float32), jnp.zeros((N, 3), jnp.float32)],
        axis=1)
    x8 = jnp.pad(x8, ((0, NPAD - N), (0, 0)))
    x_pad = jnp.pad(x, ((0, NPAD - N), (0, 0)))

    sc1 = (g1 / jnp.sqrt(1.0 + EPS)).reshape(1, 64)
    sc2 = (g2 / jnp.sqrt(1.0 + EPS)).reshape(1, 64)

    P = _sc_layer1(x8, edge_index)
    h1 = _tc_layer1(P, x_pad, W1l.T, b1l.reshape(1, 64), W1r.T,
                    sc1, be1.reshape(1, 64))
    O = _sc_layer2(h1.reshape(4 * NPAD, LANES), edge_index)
    out = _tc_layer2(O, h1, P, W2l.T, b2l.reshape(1, 64), W2r.T,
                     sc2, be2.reshape(1, 64), Wp.T, bp.reshape(1, 1))
    return out[:N].reshape(N, 1)
